# baseline (speedup 1.0000x reference)
import functools

import jax
import jax.numpy as jnp
from jax import lax
from jax.experimental import pallas as pl
from jax.experimental.pallas import tpu as pltpu
from jax.experimental.pallas import tpu_sc as plsc

VOCAB = 100000
MAX_SEQ = 8192
D_MODEL = 1024
BATCH = 4
SEQ = 8192

_INFO = plsc.get_sparse_core_info()
_NC = _INFO.num_cores
_NS = _INFO.num_subcores
_NW = _NC * _NS
_LANES = _INFO.num_lanes

POS_PER_W = SEQ // _NW
CHUNK = 8
N_CHUNKS = POS_PER_W // CHUNK
NSLICE = D_MODEL // _LANES
ROWS = BATCH * CHUNK


def _make_kernel():
    mesh = plsc.VectorSubcoreMesh(core_axis_name="c", subcore_axis_name="s")

    @functools.partial(
        pl.kernel,
        mesh=mesh,
        out_type=jax.ShapeDtypeStruct((BATCH * SEQ, D_MODEL), jnp.float32),
        scratch_types=(
            [
                pltpu.VMEM((N_CHUNKS * ROWS,), jnp.int32),
                pltpu.VMEM((ROWS, D_MODEL), jnp.float32),
                pltpu.VMEM((ROWS, D_MODEL), jnp.float32),
                pltpu.VMEM((CHUNK, D_MODEL), jnp.float32),
                pltpu.VMEM((CHUNK, D_MODEL), jnp.float32),
            ]
            + [pltpu.SemaphoreType.DMA] * 6
        ),
    )
    def emb_kernel(ids_hbm, tok_hbm, pos_hbm, out_hbm, idxp,
                   set0, set1, pos0, pos1, gsem0, gsem1, ssem0, ssem1,
                   psem0, psem1):
        sets = (set0, set1)
        posb = (pos0, pos1)
        gsem = (gsem0, gsem1)
        ssem = (ssem0, ssem1)
        psem = (psem0, psem1)
        wid = lax.axis_index("s") * _NC + lax.axis_index("c")
        pos_base = wid * POS_PER_W

        def out_view(c, b):
            return out_hbm.at[pl.ds(b * SEQ + pos_base + c * CHUNK, CHUNK)]

        def pos_view(c):
            return pos_hbm.at[pl.ds(pos_base + c * CHUNK, CHUNK)]

        def idx_view(c):
            return idxp.at[pl.ds(c * ROWS, ROWS)]

        def start_gather(c, s):
            pltpu.async_copy(tok_hbm.at[idx_view(c)], sets[s], gsem[s])

        def wait_gather(s):
            pltpu.make_async_copy(tok_hbm.at[idx_view(0)], sets[s],
                                  gsem[s]).wait()

        def start_store(c, s):
            for b in range(BATCH):
                pltpu.async_copy(sets[s].at[pl.ds(b * CHUNK, CHUNK)],
                                 out_view(c, b), ssem[s])

        def wait_store(s):
            for b in range(BATCH):
                pltpu.make_async_copy(sets[s].at[pl.ds(b * CHUNK, CHUNK)],
                                      out_view(0, 0), ssem[s]).wait()

        def start_pos(c, s):
            pltpu.async_copy(pos_view(c), posb[s], psem[s])

        def wait_pos(s):
            pltpu.make_async_copy(pos_view(0), posb[s], psem[s]).wait()

        def accumulate(s):
            def row(r, _):
                for j in range(NSLICE):
                    sl = pl.ds(j * _LANES, _LANES)
                    pv = posb[s][r, sl]
                    for b in range(BATCH):
                        row_i = b * CHUNK + r
                        sets[s][row_i, sl] = sets[s][row_i, sl] + pv
                return 0

            lax.fori_loop(0, CHUNK, row, 0)

        pltpu.sync_copy(ids_hbm.at[pl.ds(wid * N_CHUNKS * ROWS,
                                         N_CHUNKS * ROWS)], idxp)
        start_pos(0, 0)
        start_gather(0, 0)

        def group(gg, _):
            for dg in range(2):
                g = gg * 2 + dg
                s = dg
                wait_pos(s)
                if dg == 0:
                    start_pos(g + 1, 1)
                else:
                    @pl.when(gg < N_CHUNKS // 2 - 1)
                    def _():
                        start_pos(g + 1, 0)
                wait_gather(s)
                if dg == 0:
                    @pl.when(gg > 0)
                    def _():
                        wait_store(1)
                else:
                    wait_store(0)
                if dg == 0:
                    start_gather(g + 1, 1)
                else:
                    @pl.when(gg < N_CHUNKS // 2 - 1)
                    def _():
                        start_gather(g + 1, 0)
                accumulate(s)
                start_store(g, s)
            return 0

        lax.fori_loop(0, N_CHUNKS // 2, group, 0)
        wait_store(1)

    return emb_kernel


_EMB_KERNEL = _make_kernel()


@jax.jit
def kernel(token_ids, token_table, pos_table):
    ids = (token_ids.astype(jnp.int32)
           .reshape(BATCH, SEQ // CHUNK, CHUNK)
           .transpose(1, 0, 2)
           .reshape(BATCH * SEQ))
    out = _EMB_KERNEL(ids, token_table, pos_table)
    return out.reshape(BATCH, SEQ, D_MODEL)


# speedup vs baseline: 1.0197x; 1.0197x over previous
import functools

import jax
import jax.numpy as jnp
from jax import lax
from jax.experimental import pallas as pl
from jax.experimental.pallas import tpu as pltpu
from jax.experimental.pallas import tpu_sc as plsc

VOCAB = 100000
MAX_SEQ = 8192
D_MODEL = 1024
BATCH = 4
SEQ = 8192

_INFO = plsc.get_sparse_core_info()
_NC = _INFO.num_cores
_NS = _INFO.num_subcores
_NW = _NC * _NS
_LANES = _INFO.num_lanes

POS_PER_W = SEQ // _NW
CHUNK = 8
N_CHUNKS = POS_PER_W // CHUNK
NSLICE = D_MODEL // _LANES


def _make_kernel():
    mesh = plsc.VectorSubcoreMesh(core_axis_name="c", subcore_axis_name="s")

    @functools.partial(
        pl.kernel,
        mesh=mesh,
        out_type=jax.ShapeDtypeStruct((BATCH * SEQ, D_MODEL), jnp.float32),
        scratch_types=(
            [pltpu.VMEM((BATCH, POS_PER_W), jnp.int32)]
            + [pltpu.VMEM((CHUNK, D_MODEL), jnp.float32)] * (2 * BATCH + 2)
            + [pltpu.SemaphoreType.DMA] * 6
        ),
    )
    def emb_kernel(ids_hbm, tok_hbm, pos_hbm, out_hbm, idxv, *bufs_and_sems):
        rows = (bufs_and_sems[0:BATCH], bufs_and_sems[BATCH:2 * BATCH])
        posb = bufs_and_sems[2 * BATCH:2 * BATCH + 2]
        gsem = bufs_and_sems[2 * BATCH + 2:2 * BATCH + 4]
        ssem = bufs_and_sems[2 * BATCH + 4:2 * BATCH + 6]
        psem = bufs_and_sems[2 * BATCH + 6:]
        wid = lax.axis_index("s") * _NC + lax.axis_index("c")
        pos_base = wid * POS_PER_W

        def idx_view(c, b):
            return idxv.at[b, pl.ds(c * CHUNK, CHUNK)]

        def out_view(c, b):
            return out_hbm.at[pl.ds(b * SEQ + pos_base + c * CHUNK, CHUNK)]

        def pos_view(c):
            return pos_hbm.at[pl.ds(pos_base + c * CHUNK, CHUNK)]

        def start_gathers(c, s):
            for b in range(BATCH):
                pltpu.async_copy(tok_hbm.at[idx_view(c, b)], rows[s][b],
                                 gsem[s])

        def wait_gathers(s):
            for b in range(BATCH):
                pltpu.make_async_copy(tok_hbm.at[idx_view(0, 0)], rows[s][b],
                                      gsem[s]).wait()

        def start_stores(c, s):
            for b in range(BATCH):
                pltpu.async_copy(rows[s][b], out_view(c, b), ssem[s])

        def wait_stores(s):
            for b in range(BATCH):
                pltpu.make_async_copy(rows[s][b], out_view(0, 0),
                                      ssem[s]).wait()

        def start_pos(c, s):
            pltpu.async_copy(pos_view(c), posb[s], psem[s])

        def wait_pos(s):
            pltpu.make_async_copy(pos_view(0), posb[s], psem[s]).wait()

        def accumulate(s):
            def row(r, _):
                for j in range(NSLICE):
                    sl = pl.ds(j * _LANES, _LANES)
                    pv = posb[s][r, sl]
                    for b in range(BATCH):
                        rows[s][b][r, sl] = rows[s][b][r, sl] + pv
                return 0

            lax.fori_loop(0, CHUNK, row, 0)

        for b in range(BATCH):
            pltpu.async_copy(ids_hbm.at[b, pl.ds(pos_base, POS_PER_W)],
                            idxv.at[b], gsem[1])
        start_pos(0, 0)
        for b in range(BATCH):
            pltpu.make_async_copy(ids_hbm.at[0, pl.ds(pos_base, POS_PER_W)],
                                  idxv.at[b], gsem[1]).wait()
        start_gathers(0, 0)

        def group(gg, _):
            for dg in range(2):
                g = gg * 2 + dg
                s = dg
                wait_pos(s)
                if dg == 0:
                    start_pos(g + 1, 1)
                else:
                    @pl.when(gg < N_CHUNKS // 2 - 1)
                    def _():
                        start_pos(g + 1, 0)
                wait_gathers(s)
                if dg == 0:
                    @pl.when(gg > 0)
                    def _():
                        wait_stores(1)
                else:
                    wait_stores(0)
                if dg == 0:
                    start_gathers(g + 1, 1)
                else:
                    @pl.when(gg < N_CHUNKS // 2 - 1)
                    def _():
                        start_gathers(g + 1, 0)
                accumulate(s)
                start_stores(g, s)
            return 0

        lax.fori_loop(0, N_CHUNKS // 2, group, 0)
        wait_stores(1)

    return emb_kernel


_EMB_KERNEL = _make_kernel()


@jax.jit
def kernel(token_ids, token_table, pos_table):
    ids = token_ids.astype(jnp.int32)
    out = _EMB_KERNEL(ids, token_table, pos_table)
    return out.reshape(BATCH, SEQ, D_MODEL)
